# baseline (speedup 1.0000x reference)
import functools

import jax
import jax.numpy as jnp
import numpy as np
from jax import lax
from jax.experimental import pallas as pl
from jax.experimental.pallas import tpu as pltpu
from jax.experimental.pallas import tpu_sc as plsc

_FIELD_DIMS = [100000] * 26
_B = 16384
_F = 26
_FP = 32
_D = 16
_TOTAL = sum(_FIELD_DIMS)
_MLP_IN = _F * _D
_MLP_PAD = _FP * _D

_NC = 2
_NS = 16
_NW = _NC * _NS
_ROWS_W = _B // _NW
_CHUNK = 128
_GROUPS = _ROWS_W // 16
_NCHUNK = 4 * _GROUPS


_NLIN = _F * (_ROWS_W // _CHUNK)


def _sc_gather(xr, xl, emb, lin_flat):
    mesh = plsc.VectorSubcoreMesh(core_axis_name="c", subcore_axis_name="s")

    @functools.partial(
        pl.kernel,
        mesh=mesh,
        compiler_params=pltpu.CompilerParams(use_tc_tiling_on_sc=False),
        out_type=[
            jax.ShapeDtypeStruct((4, _NW * _GROUPS, _CHUNK, _D), jnp.float32),
            jax.ShapeDtypeStruct((_B,), jnp.float32),
        ],
        scratch_types=[
            pltpu.VMEM((_NCHUNK, _CHUNK), jnp.int32),
            pltpu.VMEM((_NLIN, _CHUNK), jnp.int32),
            pltpu.VMEM((_CHUNK, _D), jnp.float32),
            pltpu.VMEM((_CHUNK,), jnp.float32),
            pltpu.VMEM((_ROWS_W,), jnp.float32),
            pltpu.SemaphoreType.DMA,
            pltpu.SemaphoreType.DMA,
        ],
    )
    def gather_k(xr_hbm, xl_hbm, emb_hbm, lin_hbm, g_out, lin_out,
                 idx_v, lidx_v, rows_v, lin_v, linacc, sem_e, sem_l):
        wid = lax.axis_index("s") * _NC + lax.axis_index("c")
        pltpu.sync_copy(xr_hbm.at[wid], idx_v)
        pltpu.sync_copy(xl_hbm.at[wid], lidx_v)

        zero16 = jnp.zeros((16,), jnp.float32)

        def zbody(g, carry):
            linacc[pl.ds(g * 16, 16)] = zero16
            return carry

        lax.fori_loop(0, _ROWS_W // 16, zbody, 0)

        gbase = wid * _GROUPS

        def ebody(c, carry):
            cp_e = pltpu.async_copy(emb_hbm.at[idx_v.at[c]], rows_v, sem_e)
            cp_e.wait()
            j = c // _GROUPS
            g = lax.rem(c, _GROUPS)
            pltpu.sync_copy(rows_v, g_out.at[j, gbase + g])
            return carry

        lax.fori_loop(0, _NCHUNK, ebody, 0)

        def lbody(c, carry):
            cp_l = pltpu.async_copy(lin_hbm.at[lidx_v.at[c]], lin_v, sem_l)
            cp_l.wait()
            q = lax.rem(c, _ROWS_W // _CHUNK)
            for k in range(_CHUNK // 16):
                off = q * _CHUNK + k * 16
                linacc[pl.ds(off, 16)] = linacc[pl.ds(off, 16)] + lin_v[pl.ds(k * 16, 16)]
            return carry

        lax.fori_loop(0, _NLIN, lbody, 0)

        pltpu.sync_copy(linacc, lin_out.at[pl.ds(wid * _ROWS_W, _ROWS_W)])

    return gather_k(xr, xl, emb, lin_flat)


_BB = 512


def _tc_body(g_ref, lin_ref, s_ref, w1_ref, b1_ref, w2_ref, b2_ref, w3_ref, cb_ref, out_ref):
    g3 = g_ref[3]
    mask = lax.broadcasted_iota(jnp.int32, g3.shape, 1) < 32
    g3 = jnp.where(mask, g3, 0.0)
    e = jnp.concatenate([g_ref[0], g_ref[1], g_ref[2], g3], axis=1)
    s = jnp.dot(e, s_ref[...], preferred_element_type=jnp.float32)
    fm = 0.5 * (jnp.sum(s * s, axis=1) - jnp.sum(e * e, axis=1))
    fm_logit = jax.nn.sigmoid(fm)
    h = jnp.dot(e, w1_ref[...], preferred_element_type=jnp.float32) + b1_ref[...]
    h = jnp.maximum(h, 0.0)
    h = jnp.dot(h, w2_ref[...], preferred_element_type=jnp.float32) + b2_ref[...]
    h = jnp.maximum(h, 0.0)
    dnn = jnp.dot(h, w3_ref[...], preferred_element_type=jnp.float32)[:, 0]
    logit = lin_ref[...] + fm_logit + dnn + cb_ref[0, 0]
    out_ref[...] = jax.nn.sigmoid(logit)


def _tc_dense(g4, linsum, s_mat, W1p, b1, W2, b2, W3, cb, *, interpret=False):
    grid = (_B // _BB,)
    full = lambda shape: pl.BlockSpec(shape, lambda i: (0,) * len(shape))
    return pl.pallas_call(
        _tc_body,
        grid=grid,
        in_specs=[
            pl.BlockSpec((4, _BB, 128), lambda i: (0, i, 0)),
            pl.BlockSpec((_BB,), lambda i: (i,)),
            full((_MLP_PAD, _D)),
            full((_MLP_PAD, 256)),
            full((1, 256)),
            full((256, 128)),
            full((1, 128)),
            full((128, 1)),
            full((1, 1)),
        ],
        out_specs=pl.BlockSpec((_BB,), lambda i: (i,)),
        out_shape=jax.ShapeDtypeStruct((_B,), jnp.float32),
        interpret=interpret,
    )(g4, linsum, s_mat, W1p, b1, W2, b2, W3, cb)


_OFFSETS = np.cumsum([0] + _FIELD_DIMS[:-1]).astype(np.int32)
_S_MAT = np.concatenate(
    [np.tile(np.eye(_D, dtype=np.float32), (_F, 1)),
     np.zeros((_MLP_PAD - _MLP_IN, _D), np.float32)], axis=0)


def _reorder_indices(x):
    xi = x + _OFFSETS[None, :]
    pad = jnp.zeros((_B, _FP - _F), jnp.int32)
    xp = jnp.concatenate([xi, pad], axis=1)
    xp = xp.reshape(_NW, _GROUPS, 16, 4, 8).transpose(0, 3, 1, 2, 4)
    xr = xp.reshape(_NW, _NCHUNK, _CHUNK)
    xl = xi.reshape(_NW, _ROWS_W // _CHUNK, _CHUNK, _F).transpose(0, 3, 1, 2)
    xl = xl.reshape(_NW, _NLIN, _CHUNK)
    return xr, xl


def kernel(x, emb, lin_w, lin_b, W1, b1, W2, b2, W3, b3):
    xr, xl = _reorder_indices(x)
    g4, linsum = _sc_gather(xr, xl, emb, lin_w.reshape(-1))
    g4 = g4.reshape(4, _B, 128)
    W1p = jnp.concatenate([W1, jnp.zeros((_MLP_PAD - _MLP_IN, 256), jnp.float32)], axis=0)
    cb = (lin_b + b3).reshape(1, 1)
    return _tc_dense(
        g4, linsum, jnp.asarray(_S_MAT), W1p, b1.reshape(1, 256), W2, b2.reshape(1, 128), W3, cb
    )


# speedup vs baseline: 1.0744x; 1.0744x over previous
import functools

import jax
import jax.numpy as jnp
import numpy as np
from jax import lax
from jax.experimental import pallas as pl
from jax.experimental.pallas import tpu as pltpu
from jax.experimental.pallas import tpu_sc as plsc

_FIELD_DIMS = [100000] * 26
_B = 16384
_F = 26
_FP = 32
_D = 16
_TOTAL = sum(_FIELD_DIMS)
_MLP_IN = _F * _D
_MLP_PAD = _FP * _D

_NC = 2
_NS = 16
_NW = _NC * _NS
_ROWS_W = _B // _NW
_CHUNK = 128
_GROUPS = _ROWS_W // 16
_NCHUNK = 4 * _GROUPS


_NLIN = _F * (_ROWS_W // _CHUNK)


def _sc_gather(xr, xl, emb, lin_flat):
    mesh = plsc.VectorSubcoreMesh(core_axis_name="c", subcore_axis_name="s")

    nb_e = _NCHUNK // 8
    nb_l = _NLIN // 8

    @functools.partial(
        pl.kernel,
        mesh=mesh,
        compiler_params=pltpu.CompilerParams(use_tc_tiling_on_sc=False),
        out_type=[
            jax.ShapeDtypeStruct((4, _NW * _GROUPS, _CHUNK, _D), jnp.float32),
            jax.ShapeDtypeStruct((_B,), jnp.float32),
        ],
        scratch_types=[
            pltpu.VMEM((_NCHUNK, _CHUNK), jnp.int32),
            pltpu.VMEM((_NLIN, _CHUNK), jnp.int32),
            pltpu.VMEM((8, _CHUNK, _D), jnp.float32),
            pltpu.VMEM((8, _CHUNK, _D), jnp.float32),
            pltpu.VMEM((8, _CHUNK), jnp.float32),
            pltpu.VMEM((8, _CHUNK), jnp.float32),
            pltpu.VMEM((_ROWS_W,), jnp.float32),
            pltpu.SemaphoreType.DMA,
            pltpu.SemaphoreType.DMA,
            pltpu.SemaphoreType.DMA,
            pltpu.SemaphoreType.DMA,
        ],
    )
    def gather_k(xr_hbm, xl_hbm, emb_hbm, lin_hbm, g_out, lin_out,
                 idx_v, lidx_v, rows_a, rows_b, lbuf_a, lbuf_b, linacc,
                 sem_ea, sem_eb, sem_la, sem_lb):
        wid = lax.axis_index("s") * _NC + lax.axis_index("c")
        pltpu.sync_copy(xr_hbm.at[wid], idx_v)
        pltpu.sync_copy(xl_hbm.at[wid], lidx_v)

        zero16 = jnp.zeros((16,), jnp.float32)

        def zbody(g, carry):
            linacc[pl.ds(g * 16, 16)] = zero16
            return carry

        lax.fori_loop(0, _ROWS_W // 16, zbody, 0)

        gbase = wid * _GROUPS

        def e_dst(b):
            j = b // 4
            g0 = lax.rem(b, 4) * 8
            return g_out.at[j, pl.ds(gbase + g0, 8)]

        def e_fire(buf, sem, b):
            for i in range(8):
                pltpu.async_copy(emb_hbm.at[idx_v.at[b * 8 + i]], buf.at[i], sem)

        def e_drain(buf, sem, b):
            pltpu.make_async_copy(e_dst(b), buf, sem).wait()

        e_fire(rows_a, sem_ea, 0)

        def ebody(t, carry):
            ba = 2 * t
            bb = 2 * t + 1
            e_fire(rows_b, sem_eb, bb)
            e_drain(rows_a, sem_ea, ba)
            pltpu.sync_copy(rows_a, e_dst(ba))

            @pl.when(bb + 1 < nb_e)
            def _():
                e_fire(rows_a, sem_ea, bb + 1)

            e_drain(rows_b, sem_eb, bb)
            pltpu.sync_copy(rows_b, e_dst(bb))
            return carry

        lax.fori_loop(0, nb_e // 2, ebody, 0)

        def l_fire(buf, sem, b):
            for i in range(8):
                pltpu.async_copy(lin_hbm.at[lidx_v.at[b * 8 + i]], buf.at[i], sem)

        def l_drain(buf, sem):
            for i in range(8):
                pltpu.make_async_copy(lin_out.at[pl.ds(0, _CHUNK)], buf.at[i], sem).wait()

        def l_reduce(buf):
            for i in range(8):
                for k in range(_CHUNK // 16):
                    off = (i % 4) * _CHUNK + k * 16
                    linacc[pl.ds(off, 16)] = linacc[pl.ds(off, 16)] + buf[i, pl.ds(k * 16, 16)]

        l_fire(lbuf_a, sem_la, 0)

        def lbody(t, carry):
            ba = 2 * t
            bb = 2 * t + 1
            l_fire(lbuf_b, sem_lb, bb)
            l_drain(lbuf_a, sem_la)
            l_reduce(lbuf_a)

            @pl.when(bb + 1 < nb_l)
            def _():
                l_fire(lbuf_a, sem_la, bb + 1)

            l_drain(lbuf_b, sem_lb)
            l_reduce(lbuf_b)
            return carry

        lax.fori_loop(0, nb_l // 2, lbody, 0)

        l_drain(lbuf_a, sem_la)
        l_reduce(lbuf_a)

        pltpu.sync_copy(linacc, lin_out.at[pl.ds(wid * _ROWS_W, _ROWS_W)])

    return gather_k(xr, xl, emb, lin_flat)


_BB = 512


def _tc_body(g_ref, lin_ref, s_ref, w1_ref, b1_ref, w2_ref, b2_ref, w3_ref, cb_ref, out_ref):
    g3 = g_ref[3]
    mask = lax.broadcasted_iota(jnp.int32, g3.shape, 1) < 32
    g3 = jnp.where(mask, g3, 0.0)
    e = jnp.concatenate([g_ref[0], g_ref[1], g_ref[2], g3], axis=1)
    s = jnp.dot(e, s_ref[...], preferred_element_type=jnp.float32)
    fm = 0.5 * (jnp.sum(s * s, axis=1) - jnp.sum(e * e, axis=1))
    fm_logit = jax.nn.sigmoid(fm)
    h = jnp.dot(e, w1_ref[...], preferred_element_type=jnp.float32) + b1_ref[...]
    h = jnp.maximum(h, 0.0)
    h = jnp.dot(h, w2_ref[...], preferred_element_type=jnp.float32) + b2_ref[...]
    h = jnp.maximum(h, 0.0)
    dnn = jnp.dot(h, w3_ref[...], preferred_element_type=jnp.float32)[:, 0]
    logit = lin_ref[...] + fm_logit + dnn + cb_ref[0, 0]
    out_ref[...] = jax.nn.sigmoid(logit)


def _tc_dense(g4, linsum, s_mat, W1p, b1, W2, b2, W3, cb, *, interpret=False):
    grid = (_B // _BB,)
    full = lambda shape: pl.BlockSpec(shape, lambda i: (0,) * len(shape))
    return pl.pallas_call(
        _tc_body,
        grid=grid,
        in_specs=[
            pl.BlockSpec((4, _BB, 128), lambda i: (0, i, 0)),
            pl.BlockSpec((_BB,), lambda i: (i,)),
            full((_MLP_PAD, _D)),
            full((_MLP_PAD, 256)),
            full((1, 256)),
            full((256, 128)),
            full((1, 128)),
            full((128, 1)),
            full((1, 1)),
        ],
        out_specs=pl.BlockSpec((_BB,), lambda i: (i,)),
        out_shape=jax.ShapeDtypeStruct((_B,), jnp.float32),
        interpret=interpret,
    )(g4, linsum, s_mat, W1p, b1, W2, b2, W3, cb)


_OFFSETS = np.cumsum([0] + _FIELD_DIMS[:-1]).astype(np.int32)
_S_MAT = np.concatenate(
    [np.tile(np.eye(_D, dtype=np.float32), (_F, 1)),
     np.zeros((_MLP_PAD - _MLP_IN, _D), np.float32)], axis=0)


def _reorder_indices(x):
    xi = x + _OFFSETS[None, :]
    pad = jnp.zeros((_B, _FP - _F), jnp.int32)
    xp = jnp.concatenate([xi, pad], axis=1)
    xp = xp.reshape(_NW, _GROUPS, 16, 4, 8).transpose(0, 3, 1, 2, 4)
    xr = xp.reshape(_NW, _NCHUNK, _CHUNK)
    xl = xi.reshape(_NW, _ROWS_W // _CHUNK, _CHUNK, _F).transpose(0, 3, 1, 2)
    xl = xl.reshape(_NW, _NLIN, _CHUNK)
    return xr, xl


def kernel(x, emb, lin_w, lin_b, W1, b1, W2, b2, W3, b3):
    xr, xl = _reorder_indices(x)
    g4, linsum = _sc_gather(xr, xl, emb, lin_w.reshape(-1))
    g4 = g4.reshape(4, _B, 128)
    W1p = jnp.concatenate([W1, jnp.zeros((_MLP_PAD - _MLP_IN, 256), jnp.float32)], axis=0)
    cb = (lin_b + b3).reshape(1, 1)
    return _tc_dense(
        g4, linsum, jnp.asarray(_S_MAT), W1p, b1.reshape(1, 256), W2, b2.reshape(1, 128), W3, cb
    )
